# baseline (speedup 1.0000x reference)
import functools

import jax
import jax.numpy as jnp
import numpy as np
from jax import lax
from jax.experimental import pallas as pl
from jax.experimental.pallas import tpu as pltpu
from jax.experimental.pallas import tpu_sc as plsc

N_NODES = 10000
D = 128
NNZ = 320000
NPOS = 100000
NNEG = 100000

NC = 2
NS = 16
NW = NC * NS

N_PAD = 10240
ROWS_PT = N_PAD // NS

_mesh = plsc.VectorSubcoreMesh(
    core_axis_name="c", subcore_axis_name="s", num_cores=NC, num_subcores=NS
)

NH = D // 2
SUB_A = 80
RPC_A = 8
KA = RPC_A * SUB_A
CHUNKS_A = NNZ // KA
ITERS_A = (CHUNKS_A + NS - 1) // NS


@functools.partial(
    pl.kernel,
    out_type=jax.ShapeDtypeStruct((NC, N_PAD, NH), jnp.float32),
    mesh=_mesh,
    scratch_types=[
        pltpu.VMEM((RPC_A, SUB_A), jnp.int32),
        pltpu.VMEM((RPC_A, SUB_A), jnp.int32),
        pltpu.VMEM((KA,), jnp.float32),
        pltpu.VMEM((RPC_A, SUB_A, NH), jnp.float32),
        pltpu.VMEM_SHARED((N_PAD, NH), jnp.float32),
        pltpu.SemaphoreType.DMA((RPC_A,)),
    ],
    compiler_params=pltpu.CompilerParams(needs_layout_passes=False, use_tc_tiling_on_sc=False),
)
def _gconv_agg(feat2, src2, dst2, vals, zeros, out, idx_s, idx_d, vals_v, rows_v, agg_sh, sem):
    cid = lax.axis_index("c")
    sid = lax.axis_index("s")

    pltpu.sync_copy(
        zeros.at[pl.ds(sid * ROWS_PT, ROWS_PT)],
        agg_sh.at[pl.ds(sid * ROWS_PT, ROWS_PT)],
    )
    plsc.subcore_barrier()
    src_off = cid * N_NODES

    def chunk(it, carry):
        ch = it * NS + sid

        @pl.when(ch < CHUNKS_A)
        def _():
            row0 = ch * RPC_A
            ebase = ch * KA
            pltpu.sync_copy(src2.at[pl.ds(row0, RPC_A)], idx_s)
            pltpu.sync_copy(dst2.at[pl.ds(row0, RPC_A)], idx_d)
            pltpu.sync_copy(vals.at[pl.ds(ebase, KA)], vals_v)
            for i in range(RPC_A):
                for g in range(SUB_A // 16):
                    sl = pl.ds(g * 16, 16)
                    idx_s[i, sl] = idx_s[i, sl] + src_off
            descs = [
                pltpu.async_copy(feat2.at[idx_s.at[i]], rows_v.at[i], sem.at[i])
                for i in range(RPC_A)
            ]
            for i in range(RPC_A):
                descs[i].wait()

                def sgroup(g, c2, i=i):
                    for r in range(16):
                        row = g * 16 + r
                        bc = plsc.load_gather(
                            vals_v,
                            [jnp.full((16,), i * SUB_A + row, jnp.int32)],
                        )
                        for j in range(NH // 16):
                            rows_v[i, row, pl.ds(j * 16, 16)] = (
                                rows_v[i, row, pl.ds(j * 16, 16)] * bc
                            )
                    return c2

                lax.fori_loop(0, SUB_A // 16, sgroup, 0)
                pltpu.sync_copy(rows_v.at[i], agg_sh.at[idx_d.at[i]], add=True)

        return carry

    lax.fori_loop(0, ITERS_A, chunk, 0)
    plsc.subcore_barrier()
    pltpu.sync_copy(
        agg_sh.at[pl.ds(sid * ROWS_PT, ROWS_PT)],
        out.at[cid, pl.ds(sid * ROWS_PT, ROWS_PT)],
    )


BR = 2000


def _mm_body(a_ref, w_ref, o_ref):
    xl = a_ref[0]
    xr = a_ref[1]
    acc = jnp.dot(xl, w_ref[:NH, :], preferred_element_type=jnp.float32)
    acc = acc + jnp.dot(xr, w_ref[NH:, :], preferred_element_type=jnp.float32)
    o_ref[...] = jnp.maximum(acc, 0.0).astype(jnp.bfloat16)


def _matmul_relu(agg_parts, W):
    return pl.pallas_call(
        _mm_body,
        grid=(N_NODES // BR,),
        in_specs=[
            pl.BlockSpec((NC, BR, NH), lambda i: (0, i, 0)),
            pl.BlockSpec((D, D), lambda i: (0, 0)),
        ],
        out_specs=pl.BlockSpec((BR, D), lambda i: (i, 0)),
        out_shape=jax.ShapeDtypeStruct((N_NODES, D), jnp.bfloat16),
    )(agg_parts, W)


SUB_C = 40
RPC_C = 8
KC = RPC_C * SUB_C
HALF_PAD = 102400
NE_TOT = 2 * HALF_PAD
CHUNKS_C = NE_TOT // KC
ITERS_C = CHUNKS_C // NW


@functools.partial(
    pl.kernel,
    out_type=jax.ShapeDtypeStruct((NE_TOT, 16), jnp.float32),
    mesh=_mesh,
    scratch_types=[
        pltpu.VMEM((RPC_C, SUB_C), jnp.int32),
        pltpu.VMEM((RPC_C, SUB_C), jnp.int32),
        pltpu.VMEM((RPC_C, SUB_C, D), jnp.bfloat16),
        pltpu.VMEM((RPC_C, SUB_C, D), jnp.bfloat16),
        pltpu.VMEM((KC, 16), jnp.float32),
        pltpu.SemaphoreType.DMA((2, RPC_C)),
    ],
    compiler_params=pltpu.CompilerParams(needs_layout_passes=False, use_tc_tiling_on_sc=False),
)
def _scores(emb, e0, e1, out, idx_a, idx_b, rows_a, rows_b, sc_v, sem):
    cid = lax.axis_index("c")
    sid = lax.axis_index("s")
    wid = sid * NC + cid

    def it(i, carry):
        ch = i * NW + wid
        row0 = ch * RPC_C
        pltpu.sync_copy(e0.at[pl.ds(row0, RPC_C)], idx_a)
        pltpu.sync_copy(e1.at[pl.ds(row0, RPC_C)], idx_b)
        descs_a = [
            pltpu.async_copy(emb.at[idx_a.at[b]], rows_a.at[b], sem.at[0, b])
            for b in range(RPC_C)
        ]
        descs_b = [
            pltpu.async_copy(emb.at[idx_b.at[b]], rows_b.at[b], sem.at[1, b])
            for b in range(RPC_C)
        ]
        for b in range(RPC_C):
            descs_a[b].wait()
            descs_b[b].wait()

            def grp(g, c2, b=b):
                for k in range(8):
                    r = g * 8 + k
                    p = None
                    for j in range(D // 32):
                        a0, a1 = plsc.unpack(
                            rows_a[b, r, pl.ds(j * 32, 32)],
                            format=plsc.PackFormat.INTERLEAVED,
                        )
                        b0, b1 = plsc.unpack(
                            rows_b[b, r, pl.ds(j * 32, 32)],
                            format=plsc.PackFormat.INTERLEAVED,
                        )
                        t = a0 * b0 + a1 * b1
                        p = t if p is None else p + t
                    sc_v[b * SUB_C + r, :] = p
                return c2

            lax.fori_loop(0, SUB_C // 8, grp, 0)

        pltpu.sync_copy(sc_v, out.at[pl.ds(ch * KC, KC)])
        return carry

    lax.fori_loop(0, ITERS_C, it, 0)


ROWS2 = NE_TOT * 16 // 2048

_M = np.zeros((2048, 128), np.float32)
_M[np.arange(2048), np.arange(2048) // 16] = 1.0


def _loss_body(x_ref, m_ref, o_ref):
    s = jnp.dot(x_ref[...], m_ref[...], preferred_element_type=jnp.float32)
    r = lax.broadcasted_iota(jnp.int32, s.shape, 0)
    c = lax.broadcasted_iota(jnp.int32, s.shape, 1)
    ids = r * 128 + c
    pos_t = jnp.where(ids < NPOS, jax.nn.softplus(-s), 0.0)
    neg_t = jnp.where(
        (ids >= HALF_PAD) & (ids < HALF_PAD + NNEG), jax.nn.softplus(s), 0.0
    )
    o_ref[0, 0] = jnp.sum(pos_t + neg_t) / NPOS


def _loss(flat2):
    return pl.pallas_call(
        _loss_body,
        in_specs=[
            pl.BlockSpec((ROWS2, 2048), lambda: (0, 0)),
            pl.BlockSpec((2048, 128), lambda: (0, 0)),
        ],
        out_specs=pl.BlockSpec(memory_space=pltpu.SMEM),
        out_shape=jax.ShapeDtypeStruct((1, 1), jnp.float32),
    )(flat2, jnp.asarray(_M))


def _pad_col(col):
    return jnp.concatenate([col, jnp.zeros((HALF_PAD - NPOS,), jnp.int32)])


def kernel(features, positive_edges, negative_edges, non_zero_index, non_zero_value, W):
    src2 = non_zero_index[0].reshape(NNZ // SUB_A, SUB_A)
    dst2 = non_zero_index[1].reshape(NNZ // SUB_A, SUB_A)
    zeros = jnp.zeros((N_PAD, NH), jnp.float32)
    feat2 = jnp.concatenate([features[:, :NH], features[:, NH:]], axis=0)
    agg_parts = _gconv_agg(feat2, src2, dst2, non_zero_value, zeros)
    emb = _matmul_relu(agg_parts, W)
    e0 = jnp.concatenate(
        [_pad_col(positive_edges[:, 0]), _pad_col(negative_edges[:, 0])]
    ).reshape(NE_TOT // SUB_C, SUB_C)
    e1 = jnp.concatenate(
        [_pad_col(positive_edges[:, 1]), _pad_col(negative_edges[:, 1])]
    ).reshape(NE_TOT // SUB_C, SUB_C)
    flat = _scores(emb, e0, e1)
    return _loss(flat.reshape(ROWS2, 2048))[0, 0]


# speedup vs baseline: 1.0303x; 1.0303x over previous
import functools

import jax
import jax.numpy as jnp
import numpy as np
from jax import lax
from jax.experimental import pallas as pl
from jax.experimental.pallas import tpu as pltpu
from jax.experimental.pallas import tpu_sc as plsc

N_NODES = 10000
D = 128
NNZ = 320000
NPOS = 100000
NNEG = 100000

NC = 2
NS = 16
NW = NC * NS

N_PAD = 10240
ROWS_PT = N_PAD // NS

_mesh = plsc.VectorSubcoreMesh(
    core_axis_name="c", subcore_axis_name="s", num_cores=NC, num_subcores=NS
)

NH = D // 2
SUB_A = 128
RPC_A = 8
KA = RPC_A * SUB_A
NNZ_PAD = 327680
CHUNKS_A = NNZ_PAD // KA
ITERS_A = (CHUNKS_A + NS - 1) // NS


@functools.partial(
    pl.kernel,
    out_type=jax.ShapeDtypeStruct((NC, N_PAD, NH), jnp.float32),
    mesh=_mesh,
    scratch_types=[
        pltpu.VMEM((RPC_A, SUB_A), jnp.int32),
        pltpu.VMEM((RPC_A, SUB_A), jnp.int32),
        pltpu.VMEM((KA,), jnp.float32),
        pltpu.VMEM((RPC_A, SUB_A, NH), jnp.float32),
        pltpu.VMEM_SHARED((N_PAD, NH), jnp.float32),
        pltpu.SemaphoreType.DMA((RPC_A + 3,)),
    ],
    compiler_params=pltpu.CompilerParams(needs_layout_passes=False, use_tc_tiling_on_sc=False),
)
def _gconv_agg(feat2, src2, dst2, vals, zeros, out, idx_s, idx_d, vals_v, rows_v, agg_sh, sem):
    cid = lax.axis_index("c")
    sid = lax.axis_index("s")

    pltpu.sync_copy(
        zeros.at[pl.ds(sid * ROWS_PT, ROWS_PT)],
        agg_sh.at[pl.ds(sid * ROWS_PT, ROWS_PT)],
    )
    plsc.subcore_barrier()
    src_off = cid * N_NODES

    def chunk(it, carry):
        ch = it * NS + sid

        @pl.when(ch < CHUNKS_A)
        def _():
            row0 = ch * RPC_A
            ebase = ch * KA
            d_s = pltpu.async_copy(src2.at[pl.ds(row0, RPC_A)], idx_s, sem.at[RPC_A])
            d_d = pltpu.async_copy(dst2.at[pl.ds(row0, RPC_A)], idx_d, sem.at[RPC_A + 1])
            d_v = pltpu.async_copy(vals.at[pl.ds(ebase, KA)], vals_v, sem.at[RPC_A + 2])
            d_s.wait()
            for i in range(RPC_A):
                for g in range(SUB_A // 16):
                    sl = pl.ds(g * 16, 16)
                    idx_s[i, sl] = idx_s[i, sl] + src_off
            descs = [
                pltpu.async_copy(feat2.at[idx_s.at[i]], rows_v.at[i], sem.at[i])
                for i in range(RPC_A)
            ]
            d_v.wait()
            for dsc in descs:
                dsc.wait()

            def sub(i, c1):
                def sgroup(g, c2):
                    for r in range(16):
                        row = g * 16 + r
                        bc = plsc.load_gather(
                            vals_v,
                            [jnp.full((16,), i * SUB_A + row, jnp.int32)],
                        )
                        for j in range(NH // 16):
                            rows_v[i, row, pl.ds(j * 16, 16)] = (
                                rows_v[i, row, pl.ds(j * 16, 16)] * bc
                            )
                    return c2

                lax.fori_loop(0, SUB_A // 16, sgroup, 0)
                return c1

            lax.fori_loop(0, RPC_A, sub, 0)
            d_d.wait()
            for i in range(RPC_A):
                pltpu.sync_copy(rows_v.at[i], agg_sh.at[idx_d.at[i]], add=True)

        return carry

    lax.fori_loop(0, ITERS_A, chunk, 0)
    plsc.subcore_barrier()
    pltpu.sync_copy(
        agg_sh.at[pl.ds(sid * ROWS_PT, ROWS_PT)],
        out.at[cid, pl.ds(sid * ROWS_PT, ROWS_PT)],
    )


BR = 2000


def _mm_body(a_ref, w_ref, o_ref):
    xl = a_ref[0]
    xr = a_ref[1]
    acc = jnp.dot(xl, w_ref[:NH, :], preferred_element_type=jnp.float32)
    acc = acc + jnp.dot(xr, w_ref[NH:, :], preferred_element_type=jnp.float32)
    o_ref[...] = jnp.maximum(acc, 0.0).astype(jnp.bfloat16)


def _matmul_relu(agg_parts, W):
    return pl.pallas_call(
        _mm_body,
        grid=(N_NODES // BR,),
        in_specs=[
            pl.BlockSpec((NC, BR, NH), lambda i: (0, i, 0)),
            pl.BlockSpec((D, D), lambda i: (0, 0)),
        ],
        out_specs=pl.BlockSpec((BR, D), lambda i: (i, 0)),
        out_shape=jax.ShapeDtypeStruct((N_NODES, D), jnp.bfloat16),
    )(agg_parts, W)


SUB_C = 64
RPC_C = 8
KC = RPC_C * SUB_C
HALF_PAD = 102400
NE_TOT = 2 * HALF_PAD
CHUNKS_C = NE_TOT // KC
ITERS_C = (CHUNKS_C + NW - 1) // NW


@functools.partial(
    pl.kernel,
    out_type=jax.ShapeDtypeStruct((NE_TOT, 16), jnp.float32),
    mesh=_mesh,
    scratch_types=[
        pltpu.VMEM((RPC_C, SUB_C), jnp.int32),
        pltpu.VMEM((RPC_C, SUB_C), jnp.int32),
        pltpu.VMEM((RPC_C, SUB_C, D), jnp.bfloat16),
        pltpu.VMEM((RPC_C, SUB_C, D), jnp.bfloat16),
        pltpu.VMEM((KC, 16), jnp.float32),
        pltpu.SemaphoreType.DMA((2, RPC_C + 1)),
    ],
    compiler_params=pltpu.CompilerParams(needs_layout_passes=False, use_tc_tiling_on_sc=False),
)
def _scores(emb, e0, e1, out, idx_a, idx_b, rows_a, rows_b, sc_v, sem):
    cid = lax.axis_index("c")
    sid = lax.axis_index("s")
    wid = sid * NC + cid

    def it(i, carry):
        ch = i * NW + wid

        @pl.when(ch < CHUNKS_C)
        def _():
            row0 = ch * RPC_C
            d_a = pltpu.async_copy(e0.at[pl.ds(row0, RPC_C)], idx_a, sem.at[0, RPC_C])
            d_b = pltpu.async_copy(e1.at[pl.ds(row0, RPC_C)], idx_b, sem.at[1, RPC_C])
            d_a.wait()
            descs_a = [
                pltpu.async_copy(emb.at[idx_a.at[b]], rows_a.at[b], sem.at[0, b])
                for b in range(RPC_C)
            ]
            d_b.wait()
            descs_b = [
                pltpu.async_copy(emb.at[idx_b.at[b]], rows_b.at[b], sem.at[1, b])
                for b in range(RPC_C)
            ]
            for dsc in descs_a:
                dsc.wait()
            for dsc in descs_b:
                dsc.wait()

            def grp(g, c2):
                for k in range(16):
                    e = g * 16 + k
                    i2 = e // SUB_C
                    r = e - i2 * SUB_C
                    p = None
                    for j in range(D // 32):
                        a0, a1 = plsc.unpack(
                            rows_a[i2, r, pl.ds(j * 32, 32)],
                            format=plsc.PackFormat.INTERLEAVED,
                        )
                        b0, b1 = plsc.unpack(
                            rows_b[i2, r, pl.ds(j * 32, 32)],
                            format=plsc.PackFormat.INTERLEAVED,
                        )
                        t = a0 * b0 + a1 * b1
                        p = t if p is None else p + t
                    sc_v[e, :] = p
                return c2

            lax.fori_loop(0, KC // 16, grp, 0)
            pltpu.sync_copy(sc_v, out.at[pl.ds(ch * KC, KC)])

        return carry

    lax.fori_loop(0, ITERS_C, it, 0)


ROWS2 = NE_TOT * 16 // 2048

_M = np.zeros((2048, 128), np.float32)
_M[np.arange(2048), np.arange(2048) // 16] = 1.0


def _loss_body(x_ref, m_ref, o_ref):
    s = jnp.dot(x_ref[...], m_ref[...], preferred_element_type=jnp.float32)
    r = lax.broadcasted_iota(jnp.int32, s.shape, 0)
    c = lax.broadcasted_iota(jnp.int32, s.shape, 1)
    ids = r * 128 + c
    pos_t = jnp.where(ids < NPOS, jax.nn.softplus(-s), 0.0)
    neg_t = jnp.where(
        (ids >= HALF_PAD) & (ids < HALF_PAD + NNEG), jax.nn.softplus(s), 0.0
    )
    o_ref[0, 0] = jnp.sum(pos_t + neg_t) / NPOS


def _loss(flat2):
    return pl.pallas_call(
        _loss_body,
        in_specs=[
            pl.BlockSpec((ROWS2, 2048), lambda: (0, 0)),
            pl.BlockSpec((2048, 128), lambda: (0, 0)),
        ],
        out_specs=pl.BlockSpec(memory_space=pltpu.SMEM),
        out_shape=jax.ShapeDtypeStruct((1, 1), jnp.float32),
    )(flat2, jnp.asarray(_M))


def _pad_col(col):
    return jnp.concatenate([col, jnp.zeros((HALF_PAD - NPOS,), jnp.int32)])


def kernel(features, positive_edges, negative_edges, non_zero_index, non_zero_value, W):
    pad_e = NNZ_PAD - NNZ
    src2 = jnp.concatenate(
        [non_zero_index[0], jnp.zeros((pad_e,), jnp.int32)]
    ).reshape(NNZ_PAD // SUB_A, SUB_A)
    dst2 = jnp.concatenate(
        [non_zero_index[1], jnp.zeros((pad_e,), jnp.int32)]
    ).reshape(NNZ_PAD // SUB_A, SUB_A)
    vals_p = jnp.concatenate([non_zero_value, jnp.zeros((pad_e,), jnp.float32)])
    zeros = jnp.zeros((N_PAD, NH), jnp.float32)
    feat2 = jnp.concatenate([features[:, :NH], features[:, NH:]], axis=0)
    agg_parts = _gconv_agg(feat2, src2, dst2, vals_p, zeros)
    emb = _matmul_relu(agg_parts, W)
    e0 = jnp.concatenate(
        [_pad_col(positive_edges[:, 0]), _pad_col(negative_edges[:, 0])]
    ).reshape(NE_TOT // SUB_C, SUB_C)
    e1 = jnp.concatenate(
        [_pad_col(positive_edges[:, 1]), _pad_col(negative_edges[:, 1])]
    ).reshape(NE_TOT // SUB_C, SUB_C)
    flat = _scores(emb, e0, e1)
    return _loss(flat.reshape(ROWS2, 2048))[0, 0]


# speedup vs baseline: 1.0503x; 1.0193x over previous
import functools

import jax
import jax.numpy as jnp
import numpy as np
from jax import lax
from jax.experimental import pallas as pl
from jax.experimental.pallas import tpu as pltpu
from jax.experimental.pallas import tpu_sc as plsc

N_NODES = 10000
D = 128
NNZ = 320000
NPOS = 100000
NNEG = 100000

NC = 2
NS = 16
NW = NC * NS

N_PAD = 10240
ROWS_PT = N_PAD // NS

_mesh = plsc.VectorSubcoreMesh(
    core_axis_name="c", subcore_axis_name="s", num_cores=NC, num_subcores=NS
)

NH = D // 2
SUB_A = 128
RPC_A = 8
KA = RPC_A * SUB_A
NNZ_PAD = 327680
CHUNKS_A = NNZ_PAD // KA
ITERS_A = (CHUNKS_A + NS - 1) // NS


@functools.partial(
    pl.kernel,
    out_type=jax.ShapeDtypeStruct((NC, N_PAD, NH), jnp.float32),
    mesh=_mesh,
    scratch_types=[
        pltpu.VMEM((RPC_A, SUB_A), jnp.int32),
        pltpu.VMEM((RPC_A, SUB_A), jnp.int32),
        pltpu.VMEM((KA,), jnp.float32),
        pltpu.VMEM((RPC_A, SUB_A, NH), jnp.float32),
        pltpu.VMEM_SHARED((N_PAD, NH), jnp.float32),
        pltpu.SemaphoreType.DMA((RPC_A + 3,)),
    ],
    compiler_params=pltpu.CompilerParams(needs_layout_passes=False, use_tc_tiling_on_sc=False),
)
def _gconv_agg(feat2, src2, dst2, vals, zeros, out, idx_s, idx_d, vals_v, rows_v, agg_sh, sem):
    cid = lax.axis_index("c")
    sid = lax.axis_index("s")

    pltpu.sync_copy(
        zeros.at[pl.ds(sid * ROWS_PT, ROWS_PT)],
        agg_sh.at[pl.ds(sid * ROWS_PT, ROWS_PT)],
    )
    plsc.subcore_barrier()
    src_off = cid * N_NODES

    def chunk(it, carry):
        ch = it * NS + sid

        @pl.when(ch < CHUNKS_A)
        def _():
            row0 = ch * RPC_A
            ebase = ch * KA
            d_s = pltpu.async_copy(src2.at[pl.ds(row0, RPC_A)], idx_s, sem.at[RPC_A])
            d_d = pltpu.async_copy(dst2.at[pl.ds(row0, RPC_A)], idx_d, sem.at[RPC_A + 1])
            d_v = pltpu.async_copy(vals.at[pl.ds(ebase, KA)], vals_v, sem.at[RPC_A + 2])
            d_s.wait()
            for i in range(RPC_A):
                for g in range(SUB_A // 16):
                    sl = pl.ds(g * 16, 16)
                    idx_s[i, sl] = idx_s[i, sl] + src_off
            descs = [
                pltpu.async_copy(feat2.at[idx_s.at[i]], rows_v.at[i], sem.at[i])
                for i in range(RPC_A)
            ]
            d_v.wait()
            for dsc in descs:
                dsc.wait()

            def sub(i, c1):
                def sgroup(g, c2):
                    for r in range(16):
                        row = g * 16 + r
                        bc = plsc.load_gather(
                            vals_v,
                            [jnp.full((16,), i * SUB_A + row, jnp.int32)],
                        )
                        for j in range(NH // 16):
                            rows_v[i, row, pl.ds(j * 16, 16)] = (
                                rows_v[i, row, pl.ds(j * 16, 16)] * bc
                            )
                    return c2

                lax.fori_loop(0, SUB_A // 16, sgroup, 0)
                return c1

            lax.fori_loop(0, RPC_A, sub, 0)
            d_d.wait()
            for i in range(RPC_A):
                pltpu.sync_copy(rows_v.at[i], agg_sh.at[idx_d.at[i]], add=True)

        return carry

    lax.fori_loop(0, ITERS_A, chunk, 0)
    plsc.subcore_barrier()
    pltpu.sync_copy(
        agg_sh.at[pl.ds(sid * ROWS_PT, ROWS_PT)],
        out.at[cid, pl.ds(sid * ROWS_PT, ROWS_PT)],
    )


BR = 2000


def _mm_body(a_ref, w_ref, o_ref):
    xl = a_ref[0]
    xr = a_ref[1]
    acc = jnp.dot(xl, w_ref[:NH, :], preferred_element_type=jnp.float32)
    acc = acc + jnp.dot(xr, w_ref[NH:, :], preferred_element_type=jnp.float32)
    o_ref[...] = jnp.maximum(acc, 0.0).astype(jnp.bfloat16)


def _matmul_relu(agg_parts, W):
    return pl.pallas_call(
        _mm_body,
        grid=(N_NODES // BR,),
        in_specs=[
            pl.BlockSpec((NC, BR, NH), lambda i: (0, i, 0)),
            pl.BlockSpec((D, D), lambda i: (0, 0)),
        ],
        out_specs=pl.BlockSpec((BR, D), lambda i: (i, 0)),
        out_shape=jax.ShapeDtypeStruct((N_NODES, D), jnp.bfloat16),
    )(agg_parts, W)


SUB_C = 64
RPC_C = 8
KC = RPC_C * SUB_C
HALF_PAD = 102400
NE_TOT = 2 * HALF_PAD
CHUNKS_C = NE_TOT // KC
ITERS_C = (CHUNKS_C + NW - 1) // NW


@functools.partial(
    pl.kernel,
    out_type=jax.ShapeDtypeStruct((NE_TOT,), jnp.float32),
    mesh=_mesh,
    scratch_types=[
        pltpu.VMEM((RPC_C, SUB_C), jnp.int32),
        pltpu.VMEM((RPC_C, SUB_C), jnp.int32),
        pltpu.VMEM((RPC_C, SUB_C, D), jnp.bfloat16),
        pltpu.VMEM((RPC_C, SUB_C, D), jnp.bfloat16),
        pltpu.VMEM((256,), jnp.float32),
        pltpu.VMEM((KC,), jnp.float32),
        pltpu.SemaphoreType.DMA((2, RPC_C + 1)),
    ],
    compiler_params=pltpu.CompilerParams(needs_layout_passes=False, use_tc_tiling_on_sc=False),
)
def _scores(emb, e0, e1, out, idx_a, idx_b, rows_a, rows_b, st, scv, sem):
    cid = lax.axis_index("c")
    sid = lax.axis_index("s")
    wid = sid * NC + cid

    def it(i, carry):
        ch = i * NW + wid

        @pl.when(ch < CHUNKS_C)
        def _():
            row0 = ch * RPC_C
            d_a = pltpu.async_copy(e0.at[pl.ds(row0, RPC_C)], idx_a, sem.at[0, RPC_C])
            d_b = pltpu.async_copy(e1.at[pl.ds(row0, RPC_C)], idx_b, sem.at[1, RPC_C])
            d_a.wait()
            descs_a = [
                pltpu.async_copy(emb.at[idx_a.at[b]], rows_a.at[b], sem.at[0, b])
                for b in range(RPC_C)
            ]
            d_b.wait()
            descs_b = [
                pltpu.async_copy(emb.at[idx_b.at[b]], rows_b.at[b], sem.at[1, b])
                for b in range(RPC_C)
            ]
            for dsc in descs_a:
                dsc.wait()
            for dsc in descs_b:
                dsc.wait()

            tr_idx = jnp.arange(16, dtype=jnp.int32) * 16

            def grp(g, c2):
                for k in range(16):
                    e = g * 16 + k
                    i2 = e // SUB_C
                    r = e - i2 * SUB_C
                    p = None
                    for j in range(D // 32):
                        a0, a1 = plsc.unpack(
                            rows_a[i2, r, pl.ds(j * 32, 32)],
                            format=plsc.PackFormat.INTERLEAVED,
                        )
                        b0, b1 = plsc.unpack(
                            rows_b[i2, r, pl.ds(j * 32, 32)],
                            format=plsc.PackFormat.INTERLEAVED,
                        )
                        t = a0 * b0 + a1 * b1
                        p = t if p is None else p + t
                    st[pl.ds(k * 16, 16)] = p
                acc = None
                for j in range(16):
                    t = plsc.load_gather(st, [tr_idx + j])
                    acc = t if acc is None else acc + t
                scv[pl.ds(g * 16, 16)] = acc
                return c2

            lax.fori_loop(0, KC // 16, grp, 0)
            pltpu.sync_copy(scv, out.at[pl.ds(ch * KC, KC)])

        return carry

    lax.fori_loop(0, ITERS_C, it, 0)


ROWS2 = NE_TOT // 2048


def _loss_body(x_ref, o_ref):
    s = x_ref[...]
    r = lax.broadcasted_iota(jnp.int32, s.shape, 0)
    c = lax.broadcasted_iota(jnp.int32, s.shape, 1)
    ids = r * 2048 + c
    pos_t = jnp.where(ids < NPOS, jax.nn.softplus(-s), 0.0)
    neg_t = jnp.where(
        (ids >= HALF_PAD) & (ids < HALF_PAD + NNEG), jax.nn.softplus(s), 0.0
    )
    o_ref[0, 0] = jnp.sum(pos_t + neg_t) / NPOS


def _loss(flat2):
    return pl.pallas_call(
        _loss_body,
        in_specs=[
            pl.BlockSpec((ROWS2, 2048), lambda: (0, 0)),
        ],
        out_specs=pl.BlockSpec(memory_space=pltpu.SMEM),
        out_shape=jax.ShapeDtypeStruct((1, 1), jnp.float32),
    )(flat2)


def _pad_col(col):
    return jnp.concatenate([col, jnp.zeros((HALF_PAD - NPOS,), jnp.int32)])


def kernel(features, positive_edges, negative_edges, non_zero_index, non_zero_value, W):
    pad_e = NNZ_PAD - NNZ
    src2 = jnp.concatenate(
        [non_zero_index[0], jnp.zeros((pad_e,), jnp.int32)]
    ).reshape(NNZ_PAD // SUB_A, SUB_A)
    dst2 = jnp.concatenate(
        [non_zero_index[1], jnp.zeros((pad_e,), jnp.int32)]
    ).reshape(NNZ_PAD // SUB_A, SUB_A)
    vals_p = jnp.concatenate([non_zero_value, jnp.zeros((pad_e,), jnp.float32)])
    zeros = jnp.zeros((N_PAD, NH), jnp.float32)
    feat2 = jnp.concatenate([features[:, :NH], features[:, NH:]], axis=0)
    agg_parts = _gconv_agg(feat2, src2, dst2, vals_p, zeros)
    emb = _matmul_relu(agg_parts, W)
    e0 = jnp.concatenate(
        [_pad_col(positive_edges[:, 0]), _pad_col(negative_edges[:, 0])]
    ).reshape(NE_TOT // SUB_C, SUB_C)
    e1 = jnp.concatenate(
        [_pad_col(positive_edges[:, 1]), _pad_col(negative_edges[:, 1])]
    ).reshape(NE_TOT // SUB_C, SUB_C)
    flat = _scores(emb, e0, e1)
    return _loss(flat.reshape(ROWS2, 2048))[0, 0]
